# Initial kernel scaffold; baseline (speedup 1.0000x reference)
import jax
import jax.numpy as jnp
from jax.experimental import pallas as pl


def kernel(x_num, x_cat, tables, W1, b1, W2, b2, Ws, bs, Wt, bt):
    raise NotImplementedError("write your pallas kernel here")


# speedup vs baseline: 3.9221x; 3.9221x over previous
import functools

import jax
import jax.numpy as jnp
from jax import lax
from jax.experimental import pallas as pl
from jax.experimental.pallas import tpu as pltpu
from jax.experimental.pallas import tpu_sc as plsc

_NC = 2
_NS = 16
_NW = _NC * _NS

_IDX_LANES = 128


def _sc_gather(idx2d: jax.Array, table: jax.Array, chunk_rows: int) -> jax.Array:
    n_rows, lanes = idx2d.shape
    total = n_rows * lanes
    _, d = table.shape
    per_w = total // _NW
    assert per_w * _NW == total
    k = chunk_rows // _IDX_LANES
    n_chunks = per_w // chunk_rows
    assert n_chunks * chunk_rows == per_w
    idx_rows_per_chunk = k
    idx_rows_per_w = per_w // _IDX_LANES

    mesh = plsc.VectorSubcoreMesh(
        core_axis_name="c", subcore_axis_name="s", num_cores=_NC, num_subcores=_NS
    )

    @functools.partial(
        pl.kernel,
        mesh=mesh,
        compiler_params=pltpu.CompilerParams(use_tc_tiling_on_sc=False),
        out_type=jax.ShapeDtypeStruct((total, d), jnp.float32),
        scratch_types=[
            pltpu.VMEM((idx_rows_per_chunk, _IDX_LANES), jnp.int32),
            pltpu.VMEM((chunk_rows, d), jnp.float32),
            pltpu.SemaphoreType.DMA,
        ],
    )
    def gather_kernel(idx_hbm, table_hbm, out_hbm, idx_v, rows_v, sem):
        wid = lax.axis_index("s") * _NC + lax.axis_index("c")
        idx_row0 = wid * idx_rows_per_w
        out_row0 = wid * per_w

        def chunk_body(ci, carry):
            pltpu.sync_copy(
                idx_hbm.at[pl.ds(idx_row0 + ci * idx_rows_per_chunk, idx_rows_per_chunk)],
                idx_v,
            )
            copies = [
                pltpu.async_copy(
                    table_hbm.at[idx_v.at[j]],
                    rows_v.at[pl.ds(j * _IDX_LANES, _IDX_LANES)],
                    sem,
                )
                for j in range(k)
            ]
            for cp in copies:
                cp.wait()
            pltpu.sync_copy(
                rows_v, out_hbm.at[pl.ds(out_row0 + ci * chunk_rows, chunk_rows)]
            )
            return carry

        lax.fori_loop(0, n_chunks, chunk_body, 0)

    return gather_kernel(idx2d, table)


def _mlp_kernel(xn_ref, xe_ref, w1n_ref, w1e_ref, b1_ref, w2_ref, b2_ref,
                wst_ref, bst_ref, out_ref):
    h = jnp.dot(xn_ref[...], w1n_ref[...], preferred_element_type=jnp.float32)
    h += jnp.dot(xe_ref[...], w1e_ref[...], preferred_element_type=jnp.float32)
    h = jnp.maximum(h + b1_ref[...], 0.0)
    g = jnp.dot(h, w2_ref[...], preferred_element_type=jnp.float32)
    g = jnp.maximum(g + b2_ref[...], 0.0)
    out_ref[...] = (
        jnp.dot(g, wst_ref[...], preferred_element_type=jnp.float32) + bst_ref[...]
    )


def _tc_mlp(x_num, x_emb, w1n, w1e, b1, w2, b2, wst, bst, blk):
    b_total, num_f = x_num.shape
    fd = x_emb.shape[1]
    h = w1n.shape[1]
    h2 = w2.shape[1]
    nout = wst.shape[1]
    grid = (b_total // blk,)
    return pl.pallas_call(
        _mlp_kernel,
        grid=grid,
        in_specs=[
            pl.BlockSpec((blk, num_f), lambda i: (i, 0)),
            pl.BlockSpec((blk, fd), lambda i: (i, 0)),
            pl.BlockSpec((num_f, h), lambda i: (0, 0)),
            pl.BlockSpec((fd, h), lambda i: (0, 0)),
            pl.BlockSpec((1, h), lambda i: (0, 0)),
            pl.BlockSpec((h, h2), lambda i: (0, 0)),
            pl.BlockSpec((1, h2), lambda i: (0, 0)),
            pl.BlockSpec((h2, nout), lambda i: (0, 0)),
            pl.BlockSpec((1, nout), lambda i: (0, 0)),
        ],
        out_specs=pl.BlockSpec((blk, nout), lambda i: (i, 0)),
        out_shape=jax.ShapeDtypeStruct((b_total, nout), jnp.float32),
    )(x_num, x_emb, w1n, w1e, b1, w2, b2, wst, bst)


def kernel(x_num, x_cat, tables, W1, b1, W2, b2, Ws, bs, Wt, bt):
    b, f = x_cat.shape
    _, v, d = tables.shape
    num_f = x_num.shape[1]

    flat_table = tables.reshape(f * v, d)
    flat_idx = (x_cat + (jnp.arange(f, dtype=jnp.int32) * v)[None, :]).reshape(-1)
    idx2d = flat_idx.reshape(-1, _IDX_LANES)

    gathered = _sc_gather(idx2d, flat_table, chunk_rows=1024)
    x_emb = gathered.reshape(b, f * d)

    w1n = W1[:num_f]
    w1e = W1[num_f:]
    wst = jnp.concatenate([Ws, Wt], axis=1)
    bst = jnp.concatenate([bs, bt]).reshape(1, -1)

    out = _tc_mlp(x_num, x_emb, w1n, w1e, b1.reshape(1, -1), W2,
                  b2.reshape(1, -1), wst, bst, blk=512)
    return out[:, :1], out[:, 1:]


# speedup vs baseline: 4.1324x; 1.0536x over previous
import functools

import jax
import jax.numpy as jnp
from jax import lax
from jax.experimental import pallas as pl
from jax.experimental.pallas import tpu as pltpu
from jax.experimental.pallas import tpu_sc as plsc

_NC = 2
_NS = 16
_NW = _NC * _NS

_IDX_LANES = 128


_DPAD = 56


def _sc_gather(x_cat_t: jax.Array, tables56: jax.Array) -> jax.Array:
    f, nb = x_cat_t.shape
    _, v, dpad = tables56.shape
    assert dpad == _DPAD
    per_w_b = nb // _NW
    assert per_w_b * _NW == nb
    k = per_w_b // _IDX_LANES
    assert k * _IDX_LANES == per_w_b
    idx3 = x_cat_t.reshape(f, nb // _IDX_LANES, _IDX_LANES)

    mesh = plsc.VectorSubcoreMesh(
        core_axis_name="c", subcore_axis_name="s", num_cores=_NC, num_subcores=_NS
    )

    @functools.partial(
        pl.kernel,
        mesh=mesh,
        compiler_params=pltpu.CompilerParams(use_tc_tiling_on_sc=False),
        out_type=jax.ShapeDtypeStruct((nb, f * _DPAD), jnp.float32),
        scratch_types=[
            pltpu.VMEM((k, _IDX_LANES), jnp.int32),
            pltpu.VMEM((per_w_b, _DPAD), jnp.float32),
            pltpu.SemaphoreType.DMA,
        ],
    )
    def gather_kernel(idx_hbm, *rest):
        field_tables = rest[:f]
        out_hbm, idx_v, rows_v, sem = rest[f:]
        wid = lax.axis_index("s") * _NC + lax.axis_index("c")
        b0 = wid * per_w_b
        brow0 = wid * k

        for fi in range(f):
            pltpu.sync_copy(idx_hbm.at[fi, pl.ds(brow0, k)], idx_v)
            copies = [
                pltpu.async_copy(
                    field_tables[fi].at[idx_v.at[j]],
                    rows_v.at[pl.ds(j * _IDX_LANES, _IDX_LANES)],
                    sem,
                )
                for j in range(k)
            ]
            for cp in copies:
                cp.wait()
            pltpu.sync_copy(
                rows_v, out_hbm.at[pl.ds(b0, per_w_b), pl.ds(fi * _DPAD, _DPAD)]
            )

    return gather_kernel(idx3, *[tables56[i] for i in range(f)])


def _mlp_kernel(xn_ref, xe_ref, w1n_ref, w1e_ref, b1_ref, w2_ref, b2_ref,
                wst_ref, bst_ref, out_ref):
    h = jnp.dot(xn_ref[...], w1n_ref[...], preferred_element_type=jnp.float32)
    h += jnp.dot(xe_ref[...], w1e_ref[...], preferred_element_type=jnp.float32)
    h = jnp.maximum(h + b1_ref[...], 0.0)
    g = jnp.dot(h, w2_ref[...], preferred_element_type=jnp.float32)
    g = jnp.maximum(g + b2_ref[...], 0.0)
    out_ref[...] = (
        jnp.dot(g, wst_ref[...], preferred_element_type=jnp.float32) + bst_ref[...]
    )


def _tc_mlp(x_num, x_emb, w1n, w1e, b1, w2, b2, wst, bst, blk):
    b_total, num_f = x_num.shape
    fd = x_emb.shape[1]
    h = w1n.shape[1]
    h2 = w2.shape[1]
    nout = wst.shape[1]
    grid = (b_total // blk,)
    return pl.pallas_call(
        _mlp_kernel,
        grid=grid,
        in_specs=[
            pl.BlockSpec((blk, num_f), lambda i: (i, 0)),
            pl.BlockSpec((blk, fd), lambda i: (i, 0)),
            pl.BlockSpec((num_f, h), lambda i: (0, 0)),
            pl.BlockSpec((fd, h), lambda i: (0, 0)),
            pl.BlockSpec((1, h), lambda i: (0, 0)),
            pl.BlockSpec((h, h2), lambda i: (0, 0)),
            pl.BlockSpec((1, h2), lambda i: (0, 0)),
            pl.BlockSpec((h2, nout), lambda i: (0, 0)),
            pl.BlockSpec((1, nout), lambda i: (0, 0)),
        ],
        out_specs=pl.BlockSpec((blk, nout), lambda i: (i, 0)),
        out_shape=jax.ShapeDtypeStruct((b_total, nout), jnp.float32),
    )(x_num, x_emb, w1n, w1e, b1, w2, b2, wst, bst)


def kernel(x_num, x_cat, tables, W1, b1, W2, b2, Ws, bs, Wt, bt):
    b, f = x_cat.shape
    _, v, d = tables.shape
    num_f = x_num.shape[1]

    tables56 = jnp.pad(tables, ((0, 0), (0, 0), (0, _DPAD - d)))
    x_emb = _sc_gather(x_cat.T, tables56)

    w1n = W1[:num_f]
    w1e = jnp.pad(
        W1[num_f:].reshape(f, d, -1), ((0, 0), (0, _DPAD - d), (0, 0))
    ).reshape(f * _DPAD, -1)
    wst = jnp.concatenate([Ws, Wt], axis=1)
    bst = jnp.concatenate([bs, bt]).reshape(1, -1)

    out = _tc_mlp(x_num, x_emb, w1n, w1e, b1.reshape(1, -1), W2,
                  b2.reshape(1, -1), wst, bst, blk=512)
    return out[:, :1], out[:, 1:]


# speedup vs baseline: 4.1962x; 1.0155x over previous
import functools

import jax
import jax.numpy as jnp
from jax import lax
from jax.experimental import pallas as pl
from jax.experimental.pallas import tpu as pltpu
from jax.experimental.pallas import tpu_sc as plsc

_NC = 2
_NS = 16
_NW = _NC * _NS

_IDX_LANES = 128


_DPAD = 56


def _sc_gather(x_cat_t: jax.Array, tables56: jax.Array) -> jax.Array:
    f, nb = x_cat_t.shape
    _, v, dpad = tables56.shape
    assert dpad == _DPAD
    per_w_b = nb // _NW
    assert per_w_b * _NW == nb
    k = per_w_b // _IDX_LANES
    assert k * _IDX_LANES == per_w_b
    idx3 = x_cat_t.reshape(f, nb // _IDX_LANES, _IDX_LANES)

    mesh = plsc.VectorSubcoreMesh(
        core_axis_name="c", subcore_axis_name="s", num_cores=_NC, num_subcores=_NS
    )

    @functools.partial(
        pl.kernel,
        mesh=mesh,
        compiler_params=pltpu.CompilerParams(use_tc_tiling_on_sc=False),
        out_type=jax.ShapeDtypeStruct((nb, f * _DPAD), jnp.float32),
        scratch_types=[
            pltpu.VMEM((k, _IDX_LANES), jnp.int32),
            pltpu.VMEM((per_w_b, _DPAD), jnp.float32),
            pltpu.SemaphoreType.DMA,
        ],
    )
    def gather_kernel(idx_hbm, table_hbm, out_hbm, idx_v, rows_v, sem):
        wid = lax.axis_index("s") * _NC + lax.axis_index("c")
        b0 = wid * per_w_b
        brow0 = wid * k

        for fi in range(f):
            pltpu.sync_copy(idx_hbm.at[fi, pl.ds(brow0, k)], idx_v)
            copies = [
                pltpu.async_copy(
                    table_hbm.at[fi].at[idx_v.at[j]],
                    rows_v.at[pl.ds(j * _IDX_LANES, _IDX_LANES)],
                    sem,
                )
                for j in range(k)
            ]
            for cp in copies:
                cp.wait()
            pltpu.sync_copy(
                rows_v, out_hbm.at[pl.ds(b0, per_w_b), pl.ds(fi * _DPAD, _DPAD)]
            )

    return gather_kernel(idx3, tables56)


def _mlp_kernel(xn_ref, xe_ref, w1n_ref, w1e_ref, b1_ref, w2_ref, b2_ref,
                wst_ref, bst_ref, out_ref):
    h = jnp.dot(xn_ref[...], w1n_ref[...], preferred_element_type=jnp.float32)
    h += jnp.dot(xe_ref[...], w1e_ref[...], preferred_element_type=jnp.float32)
    h = jnp.maximum(h + b1_ref[...], 0.0)
    g = jnp.dot(h, w2_ref[...], preferred_element_type=jnp.float32)
    g = jnp.maximum(g + b2_ref[...], 0.0)
    out_ref[...] = (
        jnp.dot(g, wst_ref[...], preferred_element_type=jnp.float32) + bst_ref[...]
    )


def _tc_mlp(x_num, x_emb, w1n, w1e, b1, w2, b2, wst, bst, blk):
    b_total, num_f = x_num.shape
    fd = x_emb.shape[1]
    h = w1n.shape[1]
    h2 = w2.shape[1]
    nout = wst.shape[1]
    grid = (b_total // blk,)
    return pl.pallas_call(
        _mlp_kernel,
        grid=grid,
        in_specs=[
            pl.BlockSpec((blk, num_f), lambda i: (i, 0)),
            pl.BlockSpec((blk, fd), lambda i: (i, 0)),
            pl.BlockSpec((num_f, h), lambda i: (0, 0)),
            pl.BlockSpec((fd, h), lambda i: (0, 0)),
            pl.BlockSpec((1, h), lambda i: (0, 0)),
            pl.BlockSpec((h, h2), lambda i: (0, 0)),
            pl.BlockSpec((1, h2), lambda i: (0, 0)),
            pl.BlockSpec((h2, nout), lambda i: (0, 0)),
            pl.BlockSpec((1, nout), lambda i: (0, 0)),
        ],
        out_specs=pl.BlockSpec((blk, nout), lambda i: (i, 0)),
        out_shape=jax.ShapeDtypeStruct((b_total, nout), jnp.float32),
    )(x_num, x_emb, w1n, w1e, b1, w2, b2, wst, bst)


def kernel(x_num, x_cat, tables, W1, b1, W2, b2, Ws, bs, Wt, bt):
    b, f = x_cat.shape
    _, v, d = tables.shape
    num_f = x_num.shape[1]

    tables56 = jnp.pad(tables, ((0, 0), (0, 0), (0, _DPAD - d)))
    x_emb = _sc_gather(x_cat.T, tables56)

    w1n = W1[:num_f]
    w1e = jnp.pad(
        W1[num_f:].reshape(f, d, -1), ((0, 0), (0, _DPAD - d), (0, 0))
    ).reshape(f * _DPAD, -1)
    wst = jnp.concatenate([Ws, Wt], axis=1)
    bst = jnp.concatenate([bs, bt]).reshape(1, -1)

    out = _tc_mlp(x_num, x_emb, w1n, w1e, b1.reshape(1, -1), W2,
                  b2.reshape(1, -1), wst, bst, blk=512)
    return out[:, :1], out[:, 1:]
